# baseline (speedup 1.0000x reference)
import functools

import jax
import jax.numpy as jnp
from jax import lax
from jax.experimental import pallas as pl
from jax.experimental.pallas import tpu as pltpu
from jax.experimental.pallas import tpu_sc as plsc

_VOCAB = 1000000
_DIM = 64
_BATCH = 16384

_INFO = plsc.get_sparse_core_info()
_NC = _INFO.num_cores
_NS = _INFO.num_subcores
_LANES = _INFO.num_lanes
_NW = _NC * _NS
_BPW = _BATCH // _NW
_BL = 32
_NB = _BPW // _BL
_BG = _BL // _LANES
_SLAB = 8


def _sc_body(center_hbm, context_hbm, in_hbm, out_hbm, score_hbm,
             cidx_v, oidx_v, cbuf_v, obuf_v, score_v, sem):
    wid = lax.axis_index("s") * _NC + lax.axis_index("c")
    base = wid * _BPW

    pltpu.sync_copy(center_hbm.at[pl.ds(base, _BPW)], cidx_v)
    pltpu.sync_copy(context_hbm.at[pl.ds(base, _BPW)], oidx_v)

    lane = lax.iota(jnp.int32, _LANES)

    def fire_block(b, slot):
        for g in range(_BG):
            cv = cidx_v[pl.ds(b * _BL + g * _LANES, _LANES)]
            ov = oidx_v[pl.ds(b * _BL + g * _LANES, _LANES)]
            cs = (cv >> 3) << 3
            os_ = (ov >> 3) << 3
            for k in range(_LANES):
                kk = g * _LANES + k
                co = pl.multiple_of(cs[k], _SLAB)
                oo = pl.multiple_of(os_[k], _SLAB)
                pltpu.async_copy(in_hbm.at[:, pl.ds(co, _SLAB)],
                                 cbuf_v.at[slot, kk], sem.at[slot])
                pltpu.async_copy(out_hbm.at[:, pl.ds(oo, _SLAB)],
                                 obuf_v.at[slot, kk], sem.at[slot])

    def drain_block(slot):
        for kk in range(_BL):
            pltpu.make_async_copy(in_hbm.at[:, pl.ds(0, _SLAB)],
                                  cbuf_v.at[slot, kk], sem.at[slot]).wait()
            pltpu.make_async_copy(out_hbm.at[:, pl.ds(0, _SLAB)],
                                  obuf_v.at[slot, kk], sem.at[slot]).wait()

    def compute_block(b, slot):
        def g_body(g, carry):
            off = b * _BL + g * _LANES
            rows = g * _LANES + lane
            clane = cidx_v[pl.ds(off, _LANES)] & 7
            olane = oidx_v[pl.ds(off, _LANES)] & 7
            accs = [jnp.zeros((_LANES,), jnp.float32) for _ in range(4)]
            for d in range(_DIM):
                dcol = jnp.full((_LANES,), d, jnp.int32)
                c = plsc.load_gather(cbuf_v.at[slot], [rows, dcol, clane])
                o = plsc.load_gather(obuf_v.at[slot], [rows, dcol, olane])
                accs[d % 4] = accs[d % 4] + c * o
            score_v[pl.ds(off, _LANES)] = (
                (accs[0] + accs[1]) + (accs[2] + accs[3]))
            return carry
        lax.fori_loop(0, _BG, g_body, 0)

    fire_block(0, 0)
    fire_block(1, 1)

    def steady(b, carry):
        slot = b % 2
        drain_block(slot)
        compute_block(b - 2, slot)
        fire_block(b, slot)
        return carry

    lax.fori_loop(2, _NB, steady, 0)

    for tail in (_NB - 2, _NB - 1):
        slot = tail % 2
        drain_block(slot)
        compute_block(tail, slot)

    pltpu.sync_copy(score_v, score_hbm.at[pl.ds(base, _BPW)])


@functools.partial(
    pl.kernel,
    out_type=jax.ShapeDtypeStruct((_BATCH,), jnp.float32),
    mesh=plsc.VectorSubcoreMesh(core_axis_name="c", subcore_axis_name="s"),
    compiler_params=pltpu.CompilerParams(
        needs_layout_passes=False,
        use_tc_tiling_on_sc=False,
    ),
    scratch_types=[
        pltpu.VMEM((_BPW,), jnp.int32),
        pltpu.VMEM((_BPW,), jnp.int32),
        pltpu.VMEM((2, _BL, _DIM, _SLAB), jnp.float32),
        pltpu.VMEM((2, _BL, _DIM, _SLAB), jnp.float32),
        pltpu.VMEM((_BPW,), jnp.float32),
        pltpu.SemaphoreType.DMA((2,)),
    ],
)
def _w2v_score(center_hbm, context_hbm, in_hbm, out_hbm, score_hbm,
               cidx_v, oidx_v, cbuf_v, obuf_v, score_v, sem):
    _sc_body(center_hbm, context_hbm, in_hbm, out_hbm, score_hbm,
             cidx_v, oidx_v, cbuf_v, obuf_v, score_v, sem)


def kernel(center, context, in_embed, out_embed):
    return _w2v_score(center.astype(jnp.int32), context.astype(jnp.int32),
                      in_embed.T, out_embed.T)


# speedup vs baseline: 19.4693x; 19.4693x over previous
import functools

import jax
import jax.numpy as jnp
from jax import lax
from jax.experimental import pallas as pl
from jax.experimental.pallas import tpu as pltpu
from jax.experimental.pallas import tpu_sc as plsc

_VOCAB = 1000000
_DIM = 64
_BATCH = 16384

_INFO = plsc.get_sparse_core_info()
_NC = _INFO.num_cores
_NS = _INFO.num_subcores
_LANES = _INFO.num_lanes
_NW = _NC * _NS
_BPW = _BATCH // _NW
_GROUPS = _BPW // _LANES
_TCOL = 128


def _sc_body(center_hbm, context_hbm, in_hbm, out_hbm, score_hbm,
             cidx_v, oidx_v, cring_v, oring_v, score_v, sem):
    wid = lax.axis_index("s") * _NC + lax.axis_index("c")
    base = wid * _BPW

    pltpu.sync_copy(center_hbm.at[pl.ds(base, _BPW)], cidx_v.at[pl.ds(0, _BPW)])
    pltpu.sync_copy(context_hbm.at[pl.ds(base, _BPW)], oidx_v.at[pl.ds(0, _BPW)])

    lane = lax.iota(jnp.int32, _LANES)

    def fire(l, slot, rc, ro):
        pltpu.async_copy(in_hbm.at[:, pl.ds(rc, _TCOL)],
                         cring_v.at[slot], sem.at[slot])
        pltpu.async_copy(out_hbm.at[:, pl.ds(ro, _TCOL)],
                         oring_v.at[slot], sem.at[slot])

    def wait_slot(slot):
        pltpu.make_async_copy(in_hbm.at[:, pl.ds(0, _TCOL)],
                              cring_v.at[slot], sem.at[slot]).wait()
        pltpu.make_async_copy(out_hbm.at[:, pl.ds(0, _TCOL)],
                              oring_v.at[slot], sem.at[slot]).wait()

    def compute(j, slot):
        rc = cidx_v[pl.ds(j, _LANES)][0]
        ro = oidx_v[pl.ds(j, _LANES)][0]
        cl = jnp.full((_LANES,), rc & 127, jnp.int32)
        ol = jnp.full((_LANES,), ro & 127, jnp.int32)
        acc = jnp.zeros((_LANES,), jnp.float32)
        for chunk in range(_DIM // _LANES):
            dcol = chunk * _LANES + lane
            c = plsc.load_gather(cring_v.at[slot], [dcol, cl])
            o = plsc.load_gather(oring_v.at[slot], [dcol, ol])
            acc = acc + c * o
        s = jnp.full((_LANES,), jnp.sum(acc), jnp.float32)
        plsc.store_scatter(score_v, [jnp.full((_LANES,), j, jnp.int32)], s,
                           mask=lane == 0)

    def do_lookup(g, k):
        l = g * _LANES + k
        if isinstance(l, int) and l < 2:
            pass
        else:
            wait_slot((k - 2) % 4)
            compute(l - 2, (k - 2) % 4)
        cv = cidx_v[pl.ds(g * _LANES, _LANES)]
        ov = oidx_v[pl.ds(g * _LANES, _LANES)]
        rc = pl.multiple_of(((cv >> 7) << 7)[k], _TCOL)
        ro = pl.multiple_of(((ov >> 7) << 7)[k], _TCOL)
        fire(l, k % 4, rc, ro)

    for k in range(_LANES):
        do_lookup(0, k)

    def steady(g, carry):
        for k in range(_LANES):
            do_lookup(g, k)
        return carry

    lax.fori_loop(1, _GROUPS, steady, 0)

    for j, slot in ((_BPW - 2, (_LANES - 2) % 4), (_BPW - 1, (_LANES - 1) % 4)):
        wait_slot(slot)
        compute(j, slot)

    pltpu.sync_copy(score_v, score_hbm.at[pl.ds(base, _BPW)])


@functools.partial(
    pl.kernel,
    out_type=jax.ShapeDtypeStruct((_BATCH,), jnp.float32),
    mesh=plsc.VectorSubcoreMesh(core_axis_name="c", subcore_axis_name="s"),
    compiler_params=pltpu.CompilerParams(
        needs_layout_passes=False,
        use_tc_tiling_on_sc=True,
    ),
    scratch_types=[
        pltpu.VMEM((_BPW + _LANES,), jnp.int32),
        pltpu.VMEM((_BPW + _LANES,), jnp.int32),
        pltpu.VMEM((4, _DIM, _TCOL), jnp.float32),
        pltpu.VMEM((4, _DIM, _TCOL), jnp.float32),
        pltpu.VMEM((_BPW,), jnp.float32),
        pltpu.SemaphoreType.DMA((4,)),
    ],
)
def _w2v_score(center_hbm, context_hbm, in_hbm, out_hbm, score_hbm,
               cidx_v, oidx_v, cring_v, oring_v, score_v, sem):
    _sc_body(center_hbm, context_hbm, in_hbm, out_hbm, score_hbm,
             cidx_v, oidx_v, cring_v, oring_v, score_v, sem)


def kernel(center, context, in_embed, out_embed):
    return _w2v_score(center.astype(jnp.int32), context.astype(jnp.int32),
                      in_embed.T, out_embed.T)


# speedup vs baseline: 22.1101x; 1.1356x over previous
import functools

import jax
import jax.numpy as jnp
from jax import lax
from jax.experimental import pallas as pl
from jax.experimental.pallas import tpu as pltpu
from jax.experimental.pallas import tpu_sc as plsc

_VOCAB = 1000000
_DIM = 64
_BATCH = 16384

_INFO = plsc.get_sparse_core_info()
_NC = _INFO.num_cores
_NS = _INFO.num_subcores
_LANES = _INFO.num_lanes
_NW = _NC * _NS
_BPW = _BATCH // _NW
_GROUPS = _BPW // _LANES
_TCOL = 128


def _sc_body(center_hbm, context_hbm, in_hbm, out_hbm, score_hbm,
             cidx_v, oidx_v, cring_v, oring_v, score_v, sem):
    wid = lax.axis_index("s") * _NC + lax.axis_index("c")
    base = wid * _BPW

    pltpu.sync_copy(center_hbm.at[pl.ds(base, _BPW)], cidx_v.at[pl.ds(0, _BPW)])
    pltpu.sync_copy(context_hbm.at[pl.ds(base, _BPW)], oidx_v.at[pl.ds(0, _BPW)])

    lane = lax.iota(jnp.int32, _LANES)

    def fire(l, slot, rc, ro):
        pltpu.async_copy(in_hbm.at[:, pl.ds(rc, _TCOL)],
                         cring_v.at[slot], sem.at[slot])
        pltpu.async_copy(out_hbm.at[:, pl.ds(ro, _TCOL)],
                         oring_v.at[slot], sem.at[slot])

    def wait_slot(slot):
        pltpu.make_async_copy(in_hbm.at[:, pl.ds(0, _TCOL)],
                              cring_v.at[slot], sem.at[slot]).wait()
        pltpu.make_async_copy(out_hbm.at[:, pl.ds(0, _TCOL)],
                              oring_v.at[slot], sem.at[slot]).wait()

    def compute(j, slot):
        rc = cidx_v[pl.ds(j, _LANES)][0]
        ro = oidx_v[pl.ds(j, _LANES)][0]
        cl = jnp.full((_LANES,), rc & 127, jnp.int32)
        ol = jnp.full((_LANES,), ro & 127, jnp.int32)
        acc = jnp.zeros((_LANES,), jnp.float32)
        for chunk in range(_DIM // _LANES):
            dcol = chunk * _LANES + lane
            c = plsc.load_gather(cring_v.at[slot], [dcol, cl])
            o = plsc.load_gather(oring_v.at[slot], [dcol, ol])
            acc = acc + c * o
        s = jnp.full((_LANES,), jnp.sum(acc), jnp.float32)
        plsc.store_scatter(score_v, [jnp.full((_LANES,), j, jnp.int32)], s,
                           mask=lane == 0)

    def do_lookup(g, k):
        l = g * _LANES + k
        if isinstance(l, int) and l < 3:
            pass
        else:
            wait_slot((k - 3) % 4)
            compute(l - 3, (k - 3) % 4)
        cv = cidx_v[pl.ds(g * _LANES, _LANES)]
        ov = oidx_v[pl.ds(g * _LANES, _LANES)]
        rc = pl.multiple_of(((cv >> 7) << 7)[k], _TCOL)
        ro = pl.multiple_of(((ov >> 7) << 7)[k], _TCOL)
        fire(l, k % 4, rc, ro)

    for k in range(_LANES):
        do_lookup(0, k)

    def steady(g, carry):
        for k in range(_LANES):
            do_lookup(g, k)
        return carry

    lax.fori_loop(1, _GROUPS, steady, 0)

    for j, slot in ((_BPW - 3, (_LANES - 3) % 4),
                    (_BPW - 2, (_LANES - 2) % 4),
                    (_BPW - 1, (_LANES - 1) % 4)):
        wait_slot(slot)
        compute(j, slot)

    pltpu.sync_copy(score_v, score_hbm.at[pl.ds(base, _BPW)])


@functools.partial(
    pl.kernel,
    out_type=jax.ShapeDtypeStruct((_BATCH,), jnp.float32),
    mesh=plsc.VectorSubcoreMesh(core_axis_name="c", subcore_axis_name="s"),
    compiler_params=pltpu.CompilerParams(
        needs_layout_passes=False,
        use_tc_tiling_on_sc=True,
    ),
    scratch_types=[
        pltpu.VMEM((_BPW + _LANES,), jnp.int32),
        pltpu.VMEM((_BPW + _LANES,), jnp.int32),
        pltpu.VMEM((4, _DIM, _TCOL), jnp.float32),
        pltpu.VMEM((4, _DIM, _TCOL), jnp.float32),
        pltpu.VMEM((_BPW,), jnp.float32),
        pltpu.SemaphoreType.DMA((4,)),
    ],
)
def _w2v_score(center_hbm, context_hbm, in_hbm, out_hbm, score_hbm,
               cidx_v, oidx_v, cring_v, oring_v, score_v, sem):
    _sc_body(center_hbm, context_hbm, in_hbm, out_hbm, score_hbm,
             cidx_v, oidx_v, cring_v, oring_v, score_v, sem)


def kernel(center, context, in_embed, out_embed):
    return _w2v_score(center.astype(jnp.int32), context.astype(jnp.int32),
                      in_embed.T, out_embed.T)


# speedup vs baseline: 23.9641x; 1.0839x over previous
import functools

import jax
import jax.numpy as jnp
from jax import lax
from jax.experimental import pallas as pl
from jax.experimental.pallas import tpu as pltpu
from jax.experimental.pallas import tpu_sc as plsc

_VOCAB = 1000000
_DIM = 64
_BATCH = 16384

_INFO = plsc.get_sparse_core_info()
_NC = _INFO.num_cores
_NS = _INFO.num_subcores
_LANES = _INFO.num_lanes
_NW = _NC * _NS
_BPW = _BATCH // _NW
_GROUPS = _BPW // _LANES
_TCOL = 128


def _sc_body(center_hbm, context_hbm, in_hbm, out_hbm, score_hbm,
             cidx_v, oidx_v, cring_v, oring_v, score_v, sem):
    wid = lax.axis_index("s") * _NC + lax.axis_index("c")
    base = wid * _BPW

    pltpu.sync_copy(center_hbm.at[pl.ds(base, _BPW)], cidx_v.at[pl.ds(0, _BPW)])
    pltpu.sync_copy(context_hbm.at[pl.ds(base, _BPW)], oidx_v.at[pl.ds(0, _BPW)])

    lane = lax.iota(jnp.int32, _LANES)

    def fire(l, slot, rc, ro):
        pltpu.async_copy(in_hbm.at[:, pl.ds(rc, _TCOL)],
                         cring_v.at[slot], sem.at[slot])
        pltpu.async_copy(out_hbm.at[:, pl.ds(ro, _TCOL)],
                         oring_v.at[slot], sem.at[slot])

    def wait_slot(slot):
        pltpu.make_async_copy(in_hbm.at[:, pl.ds(0, _TCOL)],
                              cring_v.at[slot], sem.at[slot]).wait()
        pltpu.make_async_copy(out_hbm.at[:, pl.ds(0, _TCOL)],
                              oring_v.at[slot], sem.at[slot]).wait()

    def compute(j, slot):
        rc = cidx_v[pl.ds(j, _LANES)][0]
        ro = oidx_v[pl.ds(j, _LANES)][0]
        cl = jnp.full((_LANES,), rc & 127, jnp.int32)
        ol = jnp.full((_LANES,), ro & 127, jnp.int32)
        acc = jnp.zeros((_LANES,), jnp.float32)
        for chunk in range(_DIM // _LANES):
            dcol = chunk * _LANES + lane
            c = plsc.load_gather(cring_v.at[slot], [dcol, cl])
            o = plsc.load_gather(oring_v.at[slot], [dcol, ol])
            acc = acc + c * o
        s = jnp.full((_LANES,), jnp.sum(acc), jnp.float32)
        plsc.store_scatter(score_v, [jnp.full((_LANES,), j, jnp.int32)], s,
                           mask=lane == 0)

    def do_lookup(g, k):
        l = g * _LANES + k
        if isinstance(l, int) and l < 4:
            pass
        else:
            wait_slot(k % 4)
            compute(l - 4, k % 4)
        cv = cidx_v[pl.ds(g * _LANES, _LANES)]
        ov = oidx_v[pl.ds(g * _LANES, _LANES)]
        rc = pl.multiple_of(((cv >> 7) << 7)[k], _TCOL)
        ro = pl.multiple_of(((ov >> 7) << 7)[k], _TCOL)
        fire(l, k % 4, rc, ro)

    for k in range(_LANES):
        do_lookup(0, k)

    def steady(g, carry):
        for k in range(_LANES):
            do_lookup(g, k)
        return carry

    lax.fori_loop(1, _GROUPS, steady, 0)

    for j in range(_BPW - 4, _BPW):
        wait_slot(j % 4)
        compute(j, j % 4)

    pltpu.sync_copy(score_v, score_hbm.at[pl.ds(base, _BPW)])


@functools.partial(
    pl.kernel,
    out_type=jax.ShapeDtypeStruct((_BATCH,), jnp.float32),
    mesh=plsc.VectorSubcoreMesh(core_axis_name="c", subcore_axis_name="s"),
    compiler_params=pltpu.CompilerParams(
        needs_layout_passes=False,
        use_tc_tiling_on_sc=True,
    ),
    scratch_types=[
        pltpu.VMEM((_BPW + _LANES,), jnp.int32),
        pltpu.VMEM((_BPW + _LANES,), jnp.int32),
        pltpu.VMEM((4, _DIM, _TCOL), jnp.float32),
        pltpu.VMEM((4, _DIM, _TCOL), jnp.float32),
        pltpu.VMEM((_BPW,), jnp.float32),
        pltpu.SemaphoreType.DMA((4,)),
    ],
)
def _w2v_score(center_hbm, context_hbm, in_hbm, out_hbm, score_hbm,
               cidx_v, oidx_v, cring_v, oring_v, score_v, sem):
    _sc_body(center_hbm, context_hbm, in_hbm, out_hbm, score_hbm,
             cidx_v, oidx_v, cring_v, oring_v, score_v, sem)


def kernel(center, context, in_embed, out_embed):
    return _w2v_score(center.astype(jnp.int32), context.astype(jnp.int32),
                      in_embed.T, out_embed.T)


# speedup vs baseline: 26.7159x; 1.1148x over previous
import functools

import jax
import jax.numpy as jnp
from jax import lax
from jax.experimental import pallas as pl
from jax.experimental.pallas import tpu as pltpu
from jax.experimental.pallas import tpu_sc as plsc

_VOCAB = 1000000
_DIM = 64
_BATCH = 16384

_INFO = plsc.get_sparse_core_info()
_NC = _INFO.num_cores
_NS = _INFO.num_subcores
_LANES = _INFO.num_lanes
_NW = _NC * _NS
_BPW = _BATCH // _NW
_GROUPS = _BPW // _LANES
_TCOL = 128
_RING = 6


def _sc_body(center_hbm, context_hbm, in_hbm, out_hbm, score_hbm,
             cidx_v, oidx_v, cring_v, oring_v, score_v, sem):
    wid = lax.axis_index("s") * _NC + lax.axis_index("c")
    base = wid * _BPW

    pltpu.sync_copy(center_hbm.at[pl.ds(base, _BPW)], cidx_v.at[pl.ds(0, _BPW)])
    pltpu.sync_copy(context_hbm.at[pl.ds(base, _BPW)], oidx_v.at[pl.ds(0, _BPW)])

    lane = lax.iota(jnp.int32, _LANES)

    def fire(l, slot, rc, ro):
        pltpu.async_copy(in_hbm.at[:, pl.ds(rc, _TCOL)],
                         cring_v.at[slot], sem.at[slot])
        pltpu.async_copy(out_hbm.at[:, pl.ds(ro, _TCOL)],
                         oring_v.at[slot], sem.at[slot])

    def wait_slot(slot):
        pltpu.make_async_copy(in_hbm.at[:, pl.ds(0, _TCOL)],
                              cring_v.at[slot], sem.at[slot]).wait()
        pltpu.make_async_copy(out_hbm.at[:, pl.ds(0, _TCOL)],
                              oring_v.at[slot], sem.at[slot]).wait()

    def compute(j, slot):
        rc = cidx_v[pl.ds(j, _LANES)][0]
        ro = oidx_v[pl.ds(j, _LANES)][0]
        cl = jnp.full((_LANES,), rc & 127, jnp.int32)
        ol = jnp.full((_LANES,), ro & 127, jnp.int32)
        acc = jnp.zeros((_LANES,), jnp.float32)
        for chunk in range(_DIM // _LANES):
            dcol = chunk * _LANES + lane
            c = plsc.load_gather(cring_v.at[slot], [dcol, cl])
            o = plsc.load_gather(oring_v.at[slot], [dcol, ol])
            acc = acc + c * o
        s = jnp.full((_LANES,), jnp.sum(acc), jnp.float32)
        plsc.store_scatter(score_v, [jnp.full((_LANES,), j, jnp.int32)], s,
                           mask=lane == 0)

    def do_lookup(g, k):
        l = g * _LANES + k
        if isinstance(l, int) and l < _RING:
            pass
        else:
            j = l - _RING
            jslot = lax.rem(j, _RING)
            wait_slot(jslot)
            compute(j, jslot)
        cv = cidx_v[pl.ds(g * _LANES, _LANES)]
        ov = oidx_v[pl.ds(g * _LANES, _LANES)]
        rc = pl.multiple_of(((cv >> 7) << 7)[k], _TCOL)
        ro = pl.multiple_of(((ov >> 7) << 7)[k], _TCOL)
        fire(l, lax.rem(jnp.int32(l), _RING) if not isinstance(l, int) else l % _RING, rc, ro)

    for k in range(_LANES):
        do_lookup(0, k)

    def steady(g, carry):
        for k in range(_LANES):
            do_lookup(g, k)
        return carry

    lax.fori_loop(1, _GROUPS, steady, 0)

    for j in range(_BPW - _RING, _BPW):
        wait_slot(j % _RING)
        compute(j, j % _RING)

    pltpu.sync_copy(score_v, score_hbm.at[pl.ds(base, _BPW)])


@functools.partial(
    pl.kernel,
    out_type=jax.ShapeDtypeStruct((_BATCH,), jnp.float32),
    mesh=plsc.VectorSubcoreMesh(core_axis_name="c", subcore_axis_name="s"),
    compiler_params=pltpu.CompilerParams(
        needs_layout_passes=False,
        use_tc_tiling_on_sc=True,
    ),
    scratch_types=[
        pltpu.VMEM((_BPW + _LANES,), jnp.int32),
        pltpu.VMEM((_BPW + _LANES,), jnp.int32),
        pltpu.VMEM((_RING, _DIM, _TCOL), jnp.float32),
        pltpu.VMEM((_RING, _DIM, _TCOL), jnp.float32),
        pltpu.VMEM((_BPW,), jnp.float32),
        pltpu.SemaphoreType.DMA((_RING,)),
    ],
)
def _w2v_score(center_hbm, context_hbm, in_hbm, out_hbm, score_hbm,
               cidx_v, oidx_v, cring_v, oring_v, score_v, sem):
    _sc_body(center_hbm, context_hbm, in_hbm, out_hbm, score_hbm,
             cidx_v, oidx_v, cring_v, oring_v, score_v, sem)


def kernel(center, context, in_embed, out_embed):
    return _w2v_score(center.astype(jnp.int32), context.astype(jnp.int32),
                      in_embed.T, out_embed.T)


# speedup vs baseline: 26.8257x; 1.0041x over previous
import functools

import jax
import jax.numpy as jnp
from jax import lax
from jax.experimental import pallas as pl
from jax.experimental.pallas import tpu as pltpu
from jax.experimental.pallas import tpu_sc as plsc

_VOCAB = 1000000
_DIM = 64
_BATCH = 16384

_INFO = plsc.get_sparse_core_info()
_NC = _INFO.num_cores
_NS = _INFO.num_subcores
_LANES = _INFO.num_lanes
_NW = _NC * _NS
_BPW = _BATCH // _NW
_GROUPS = _BPW // _LANES
_TCOL = 128
_RING = 7


def _sc_body(center_hbm, context_hbm, in_hbm, out_hbm, score_hbm,
             cidx_v, oidx_v, cring_v, oring_v, score_v, sem):
    wid = lax.axis_index("s") * _NC + lax.axis_index("c")
    base = wid * _BPW

    pltpu.sync_copy(center_hbm.at[pl.ds(base, _BPW)], cidx_v.at[pl.ds(0, _BPW)])
    pltpu.sync_copy(context_hbm.at[pl.ds(base, _BPW)], oidx_v.at[pl.ds(0, _BPW)])

    lane = lax.iota(jnp.int32, _LANES)

    def fire(l, slot, rc, ro):
        pltpu.async_copy(in_hbm.at[:, pl.ds(rc, _TCOL)],
                         cring_v.at[slot], sem.at[slot])
        pltpu.async_copy(out_hbm.at[:, pl.ds(ro, _TCOL)],
                         oring_v.at[slot], sem.at[slot])

    def wait_slot(slot):
        pltpu.make_async_copy(in_hbm.at[:, pl.ds(0, _TCOL)],
                              cring_v.at[slot], sem.at[slot]).wait()
        pltpu.make_async_copy(out_hbm.at[:, pl.ds(0, _TCOL)],
                              oring_v.at[slot], sem.at[slot]).wait()

    def compute(j, slot):
        rc = cidx_v[pl.ds(j, _LANES)][0]
        ro = oidx_v[pl.ds(j, _LANES)][0]
        cl = jnp.full((_LANES,), rc & 127, jnp.int32)
        ol = jnp.full((_LANES,), ro & 127, jnp.int32)
        acc = jnp.zeros((_LANES,), jnp.float32)
        for chunk in range(_DIM // _LANES):
            dcol = chunk * _LANES + lane
            c = plsc.load_gather(cring_v.at[slot], [dcol, cl])
            o = plsc.load_gather(oring_v.at[slot], [dcol, ol])
            acc = acc + c * o
        s = jnp.full((_LANES,), jnp.sum(acc), jnp.float32)
        plsc.store_scatter(score_v, [jnp.full((_LANES,), j, jnp.int32)], s,
                           mask=lane == 0)

    def do_lookup(g, k):
        l = g * _LANES + k
        if isinstance(l, int) and l < _RING:
            pass
        else:
            j = l - _RING
            jslot = lax.rem(j, _RING)
            wait_slot(jslot)
            compute(j, jslot)
        cv = cidx_v[pl.ds(g * _LANES, _LANES)]
        ov = oidx_v[pl.ds(g * _LANES, _LANES)]
        rc = pl.multiple_of(((cv >> 7) << 7)[k], _TCOL)
        ro = pl.multiple_of(((ov >> 7) << 7)[k], _TCOL)
        fire(l, lax.rem(jnp.int32(l), _RING) if not isinstance(l, int) else l % _RING, rc, ro)

    for k in range(_LANES):
        do_lookup(0, k)

    def steady(g, carry):
        for k in range(_LANES):
            do_lookup(g, k)
        return carry

    lax.fori_loop(1, _GROUPS, steady, 0)

    for j in range(_BPW - _RING, _BPW):
        wait_slot(j % _RING)
        compute(j, j % _RING)

    pltpu.sync_copy(score_v, score_hbm.at[pl.ds(base, _BPW)])


@functools.partial(
    pl.kernel,
    out_type=jax.ShapeDtypeStruct((_BATCH,), jnp.float32),
    mesh=plsc.VectorSubcoreMesh(core_axis_name="c", subcore_axis_name="s"),
    compiler_params=pltpu.CompilerParams(
        needs_layout_passes=False,
        use_tc_tiling_on_sc=True,
    ),
    scratch_types=[
        pltpu.VMEM((_BPW + _LANES,), jnp.int32),
        pltpu.VMEM((_BPW + _LANES,), jnp.int32),
        pltpu.VMEM((_RING, _DIM, _TCOL), jnp.float32),
        pltpu.VMEM((_RING, _DIM, _TCOL), jnp.float32),
        pltpu.VMEM((_BPW,), jnp.float32),
        pltpu.SemaphoreType.DMA((_RING,)),
    ],
)
def _w2v_score(center_hbm, context_hbm, in_hbm, out_hbm, score_hbm,
               cidx_v, oidx_v, cring_v, oring_v, score_v, sem):
    _sc_body(center_hbm, context_hbm, in_hbm, out_hbm, score_hbm,
             cidx_v, oidx_v, cring_v, oring_v, score_v, sem)


def kernel(center, context, in_embed, out_embed):
    return _w2v_score(center.astype(jnp.int32), context.astype(jnp.int32),
                      in_embed.T, out_embed.T)
